# baseline (speedup 1.0000x reference)
import functools

import jax
import jax.numpy as jnp
from jax import lax
from jax.experimental import pallas as pl
from jax.experimental.pallas import tpu as pltpu
from jax.experimental.pallas import tpu_sc as plsc


def _silu(x):
    return x * jax.nn.sigmoid(x)


def _router_body(x_ref, gate_ref, meta_ref, tot_ref, wcols_ref, *, M, E):
    xb16 = x_ref[...].astype(jnp.bfloat16)
    g16 = gate_ref[...].astype(jnp.bfloat16)
    logits = jax.lax.dot_general(
        g16, xb16, (((1,), (1,)), ((), ())),
        preferred_element_type=jnp.float32)
    m = jnp.max(logits, axis=0, keepdims=True)
    ex = jnp.exp(logits - m)
    w = ex / jnp.sum(ex, axis=0, keepdims=True)
    iota = jax.lax.broadcasted_iota(jnp.int32, w.shape, 0)
    m1 = jnp.max(w, axis=0, keepdims=True)
    i1 = jnp.min(jnp.where(w == m1, iota, E), axis=0, keepdims=True)
    wm = jnp.where(iota == i1, -1.0, w)
    m2 = jnp.max(wm, axis=0, keepdims=True)
    i2 = jnp.min(jnp.where(wm == m2, iota, E), axis=0, keepdims=True)
    keep = (iota == i1) | (iota == i2)
    counts = keep.astype(jnp.float32)
    g = counts
    k = 1
    while k < M:
        shifted = jnp.concatenate(
            [jnp.zeros((E, k), jnp.float32), g[:, :M - k]], axis=1)
        g = g + shifted
        k *= 2
    rank = g - counts
    tot = g[:, M - 1:M]
    o = tot
    k = 1
    while k < E:
        o = o + jnp.concatenate(
            [jnp.zeros((k, 1), jnp.float32), o[:E - k, :]], axis=0)
        k *= 2
    offs = o - tot
    pos = offs + rank
    p1 = jnp.sum(jnp.where(iota == i1, pos, 0.0), axis=0, keepdims=True)
    p2 = jnp.sum(jnp.where(iota == i2, pos, 0.0), axis=0, keepdims=True)
    meta_ref[...] = jnp.concatenate([p1, p2, m1, m2], axis=0)
    tot_ref[...] = tot
    logits2 = jax.lax.dot_general(
        xb16, g16, (((1,), (1,)), ((), ())),
        preferred_element_type=jnp.float32)
    mm = jnp.max(logits2, axis=1, keepdims=True)
    ex2 = jnp.exp(logits2 - mm)
    w2o = ex2 / jnp.sum(ex2, axis=1, keepdims=True)
    iota2 = jax.lax.broadcasted_iota(jnp.int32, w2o.shape, 1)
    n1 = jnp.max(w2o, axis=1, keepdims=True)
    j1 = jnp.min(jnp.where(w2o == n1, iota2, E), axis=1, keepdims=True)
    wn = jnp.where(iota2 == j1, -1.0, w2o)
    n2 = jnp.max(wn, axis=1, keepdims=True)
    wcols_ref[...] = jnp.concatenate(
        [n1, n2, n1, n2, n1, n2, n1, n2], axis=1)


def _shared_body(x_ref, sgw_ref, wg_ref, wu_ref, wd_ref, out_ref,
                 xb16_ref, sg_ref, *, M, TS):
    st = pl.program_id(0)

    @pl.when(st == 0)
    def _prep():
        xb16_ref[...] = x_ref[...].astype(jnp.bfloat16)
        sgw16 = sgw_ref[...].astype(jnp.bfloat16).astype(jnp.float32)
        sgl = jnp.sum(xb16_ref[...].astype(jnp.float32) * sgw16, axis=-1,
                      keepdims=True)
        sg_ref[...] = jnp.broadcast_to(jax.nn.sigmoid(sgl), (M, 8))

    xb16 = xb16_ref[...]
    wg = wg_ref[...].astype(jnp.bfloat16)
    wu = wu_ref[...].astype(jnp.bfloat16)
    gs = jax.lax.dot_general(xb16, wg, (((1,), (1,)), ((), ())),
                             preferred_element_type=jnp.float32)
    us = jax.lax.dot_general(xb16, wu, (((1,), (1,)), ((), ())),
                             preferred_element_type=jnp.float32)
    sa = (_silu(gs) * us).astype(jnp.bfloat16)
    wd = wd_ref[...].astype(jnp.bfloat16)
    so = jax.lax.dot_general(sa, wd, (((1,), (1,)), ((), ())),
                             preferred_element_type=jnp.float32)
    acc = so * sg_ref[:, 0:1]

    @pl.when(st == 0)
    def _init():
        out_ref[...] = acc

    @pl.when(st != 0)
    def _accum():
        out_ref[...] = out_ref[...] + acc


def _expert_body(tid_ref, eid_ref, fst_ref, vld_ref,
                 xs_ref, w13_ref, w2_ref, cumtot_ref, eo_ref, *, R, E, I):
    t = pl.program_id(0)

    @pl.when(vld_ref[t] == 1)
    def _work():
        e = eid_ref[t]
        tile = tid_ref[t]
        xt = xs_ref[...].astype(jnp.bfloat16)
        h = jax.lax.dot_general(xt, w13_ref[0], (((1,), (1,)), ((), ())),
                                preferred_element_type=jnp.float32)
        act = (_silu(h[:, :I]) * h[:, I:]).astype(jnp.bfloat16)
        row_g = (tile * R + jax.lax.broadcasted_iota(
            jnp.int32, (R, 1), 0)).astype(jnp.float32)
        rexp = jnp.sum((row_g >= cumtot_ref[...]).astype(jnp.float32),
                       axis=1, keepdims=True)
        mask = rexp == e.astype(jnp.float32)
        act = jnp.where(mask, act, jnp.bfloat16(0))
        eo = jax.lax.dot_general(act, w2_ref[0], (((1,), (1,)), ((), ())),
                                 preferred_element_type=jnp.float32)

        @pl.when(fst_ref[t] == 1)
        def _init():
            eo_ref[...] = eo

        @pl.when(fst_ref[t] == 0)
        def _accum():
            eo_ref[...] = eo_ref[...] + eo


def _make_dispatch(M, H, P, NW, TPW):
    mesh = plsc.VectorSubcoreMesh(core_axis_name="c", subcore_axis_name="s",
                                  num_cores=2, num_subcores=16)

    @functools.partial(
        pl.kernel, mesh=mesh,
        out_type=jax.ShapeDtypeStruct((P, H), jnp.float32),
        scratch_types=[
            pltpu.VMEM((TPW, H), jnp.float32),
            pltpu.VMEM((TPW,), jnp.float32),
            pltpu.VMEM((TPW,), jnp.float32),
            pltpu.VMEM((TPW,), jnp.int32),
            pltpu.VMEM((TPW,), jnp.int32),
            pltpu.SemaphoreType.DMA,
        ],
    )
    def dispatch(x_hbm, meta_hbm, xs_hbm, xrows, p1f, p2f, p1i, p2i, sem):
        wid = lax.axis_index("s") * 2 + lax.axis_index("c")
        base = wid * TPW
        pltpu.sync_copy(x_hbm.at[pl.ds(base, TPW)], xrows)
        pltpu.sync_copy(meta_hbm.at[0, pl.ds(base, TPW)], p1f)
        pltpu.sync_copy(meta_hbm.at[1, pl.ds(base, TPW)], p2f)
        for j in range(TPW // 16):
            s = pl.ds(j * 16, 16)
            p1i[s] = p1f[s].astype(jnp.int32)
            p2i[s] = p2f[s].astype(jnp.int32)
        pltpu.async_copy(xrows, xs_hbm.at[p1i], sem).wait()
        pltpu.async_copy(xrows, xs_hbm.at[p2i], sem).wait()

    return dispatch


def _make_gather2(M, H, P, NW, TPW):
    mesh = plsc.VectorSubcoreMesh(core_axis_name="c", subcore_axis_name="s",
                                  num_cores=2, num_subcores=16)

    @functools.partial(
        pl.kernel, mesh=mesh,
        out_type=[jax.ShapeDtypeStruct((M, H), jnp.float32),
                  jax.ShapeDtypeStruct((M, H), jnp.float32)],
        scratch_types=[
            pltpu.VMEM((TPW, H), jnp.float32),
            pltpu.VMEM((TPW,), jnp.float32),
            pltpu.VMEM((TPW,), jnp.int32),
            pltpu.SemaphoreType.DMA,
        ],
    )
    def gather2(eo_hbm, meta_hbm, e1_hbm, e2_hbm, rows, pf, pi, sem):
        wid = lax.axis_index("s") * 2 + lax.axis_index("c")
        base = wid * TPW
        pltpu.sync_copy(meta_hbm.at[0, pl.ds(base, TPW)], pf)
        for j in range(TPW // 16):
            sl = pl.ds(j * 16, 16)
            pi[sl] = pf[sl].astype(jnp.int32)
        pltpu.async_copy(eo_hbm.at[pi], rows, sem).wait()
        pltpu.sync_copy(rows, e1_hbm.at[pl.ds(base, TPW)])
        pltpu.sync_copy(meta_hbm.at[1, pl.ds(base, TPW)], pf)
        for j in range(TPW // 16):
            sl = pl.ds(j * 16, 16)
            pi[sl] = pf[sl].astype(jnp.int32)
        pltpu.async_copy(eo_hbm.at[pi], rows, sem).wait()
        pltpu.sync_copy(rows, e2_hbm.at[pl.ds(base, TPW)])

    return gather2


def _final_body(sh_ref, e1_ref, e2_ref, wc_ref, out_ref):
    w1 = wc_ref[:, 0:1]
    w2 = wc_ref[:, 1:2]
    out_ref[...] = sh_ref[...] + w1 * e1_ref[...] + w2 * e2_ref[...]


def kernel(hidden_states, w13_stacked, w2_stacked, gate_w,
           shared_expert_gate_w, shared_gate_up_w, shared_down_w):
    orig_shape = hidden_states.shape
    H = orig_shape[-1]
    x = hidden_states.reshape(-1, H)
    M = x.shape[0]
    E, twoI, _ = w13_stacked.shape
    I = twoI // 2
    S = shared_down_w.shape[1]
    P = 2 * M
    R = 256
    NT = P // R
    VISITS = NT + E - 1
    NW = 32
    TPW = M // NW

    meta, tot_f, wcols = pl.pallas_call(
        functools.partial(_router_body, M=M, E=E),
        grid=(1,),
        in_specs=[
            pl.BlockSpec((M, H), lambda i: (0, 0)),
            pl.BlockSpec((E, H), lambda i: (0, 0)),
        ],
        out_specs=[
            pl.BlockSpec((4, M), lambda i: (0, 0)),
            pl.BlockSpec((E, 1), lambda i: (0, 0)),
            pl.BlockSpec((M, 8), lambda i: (0, 0)),
        ],
        out_shape=[
            jax.ShapeDtypeStruct((4, M), jnp.float32),
            jax.ShapeDtypeStruct((E, 1), jnp.float32),
            jax.ShapeDtypeStruct((M, 8), jnp.float32),
        ],
    )(x, gate_w)

    tot = tot_f.reshape(E).astype(jnp.int32)
    cumtot = jnp.cumsum(tot)
    starts = cumtot - tot
    tl = jnp.arange(NT, dtype=jnp.int32) * R
    inter = ((starts[None, :] < tl[:, None] + R)
             & (cumtot[None, :] > tl[:, None])
             & (tot[None, :] > 0))
    flat = inter.reshape(-1)
    nvis = jnp.sum(flat.astype(jnp.int32))
    e_last = jnp.max(jnp.where(tot > 0, jnp.arange(E), -1))
    fill = (NT - 1) * E + e_last
    idx0 = jnp.nonzero(flat, size=VISITS, fill_value=0)[0].astype(jnp.int32)
    validv = (jnp.arange(VISITS) < nvis).astype(jnp.int32)
    idx = jnp.where(validv == 1, idx0, fill).astype(jnp.int32)
    tile_id = idx // E
    exp_id = idx % E
    prev_tile = jnp.concatenate([jnp.full((1,), -1, jnp.int32), tile_id[:-1]])
    first = ((tile_id != prev_tile) & (validv == 1)).astype(jnp.int32)
    cumtot_f = cumtot.astype(jnp.float32).reshape(1, E)

    xs = _make_dispatch(M, H, P, NW, TPW)(x, meta)

    TS = 512
    shared = pl.pallas_call(
        functools.partial(_shared_body, M=M, TS=TS),
        grid=(S // TS,),
        in_specs=[
            pl.BlockSpec((M, H), lambda st: (0, 0)),
            pl.BlockSpec((1, H), lambda st: (0, 0)),
            pl.BlockSpec((TS, H), lambda st: (st, 0)),
            pl.BlockSpec((TS, H), lambda st: (st + S // TS, 0)),
            pl.BlockSpec((H, TS), lambda st: (0, st)),
        ],
        out_specs=pl.BlockSpec((M, H), lambda st: (0, 0)),
        out_shape=jax.ShapeDtypeStruct((M, H), jnp.float32),
        scratch_shapes=[
            pltpu.VMEM((M, H), jnp.bfloat16),
            pltpu.VMEM((M, 8), jnp.float32),
        ],
        compiler_params=pltpu.CompilerParams(
            dimension_semantics=("arbitrary",)),
    )(x, shared_expert_gate_w, shared_gate_up_w, shared_gate_up_w,
      shared_down_w)

    eo = pl.pallas_call(
        functools.partial(_expert_body, R=R, E=E, I=I),
        grid_spec=pltpu.PrefetchScalarGridSpec(
            num_scalar_prefetch=4,
            grid=(VISITS,),
            in_specs=[
                pl.BlockSpec((R, H), lambda t, tid, eid, fst, vld:
                             (tid[t], 0)),
                pl.BlockSpec((1, twoI, H), lambda t, tid, eid, fst, vld:
                             (eid[t], 0, 0)),
                pl.BlockSpec((1, H, I), lambda t, tid, eid, fst, vld:
                             (eid[t], 0, 0)),
                pl.BlockSpec((1, E), lambda t, tid, eid, fst, vld: (0, 0)),
            ],
            out_specs=pl.BlockSpec((R, H), lambda t, tid, eid, fst, vld:
                                   (tid[t], 0)),
        ),
        out_shape=jax.ShapeDtypeStruct((P, H), jnp.float32),
        compiler_params=pltpu.CompilerParams(
            dimension_semantics=("arbitrary",)),
    )(tile_id, exp_id, first, validv, xs, w13_stacked, w2_stacked, cumtot_f)

    e1_tok, e2_tok = _make_gather2(M, H, P, NW, TPW)(eo, meta)

    TF = 512
    out = pl.pallas_call(
        _final_body,
        grid=(M // TF,),
        in_specs=[
            pl.BlockSpec((TF, H), lambda i: (i, 0)),
            pl.BlockSpec((TF, H), lambda i: (i, 0)),
            pl.BlockSpec((TF, H), lambda i: (i, 0)),
            pl.BlockSpec((TF, 8), lambda i: (i, 0)),
        ],
        out_specs=pl.BlockSpec((TF, H), lambda i: (i, 0)),
        out_shape=jax.ShapeDtypeStruct((M, H), jnp.float32),
        compiler_params=pltpu.CompilerParams(
            dimension_semantics=("arbitrary",)),
    )(shared, e1_tok, e2_tok, wcols)
    return out.reshape(orig_shape)
